# baseline (speedup 1.0000x reference)
import jax
import jax.numpy as jnp
from jax.experimental import pallas as pl
from jax.experimental.pallas import tpu as pltpu

DEPTH_ = 1000
ROWS_PER_BLOCK = 512


def _onehot_body(idx_ref, out_ref):
    idx = idx_ref[0]
    iota = jax.lax.broadcasted_iota(jnp.int32, (ROWS_PER_BLOCK, DEPTH_), 1)
    out_ref[...] = (iota == idx).astype(jnp.float32)


def kernel(indices):
    b, f = indices.shape
    n = b * f
    num_blocks = n // ROWS_PER_BLOCK
    idx3 = indices.reshape(num_blocks, ROWS_PER_BLOCK, 1).astype(jnp.int32)
    out = pl.pallas_call(
        _onehot_body,
        grid=(num_blocks,),
        in_specs=[pl.BlockSpec((1, ROWS_PER_BLOCK, 1), lambda i: (i, 0, 0))],
        out_specs=pl.BlockSpec((ROWS_PER_BLOCK, DEPTH_), lambda i: (i, 0)),
        out_shape=jax.ShapeDtypeStruct((n, DEPTH_), jnp.float32),
    )(idx3)
    return out.reshape(b, f, DEPTH_)


# speedup vs baseline: 1.3988x; 1.3988x over previous
import jax
import jax.numpy as jnp
from jax.experimental import pallas as pl
from jax.experimental.pallas import tpu as pltpu

DEPTH_ = 1000
ROWS_PER_BLOCK = 32


def _onehot_body(idx_ref, out_ref):
    r, f, _ = idx_ref.shape
    iota = jax.lax.broadcasted_iota(jnp.int32, (r, f, DEPTH_), 2)
    out_ref[...] = (iota == idx_ref[...]).astype(jnp.float32)


def kernel(indices):
    b, f = indices.shape
    idx3 = indices.astype(jnp.int32)[..., None]
    out = pl.pallas_call(
        _onehot_body,
        grid=(b // ROWS_PER_BLOCK,),
        in_specs=[pl.BlockSpec((ROWS_PER_BLOCK, f, 1), lambda i: (i, 0, 0))],
        out_specs=pl.BlockSpec((ROWS_PER_BLOCK, f, DEPTH_), lambda i: (i, 0, 0)),
        out_shape=jax.ShapeDtypeStruct((b, f, DEPTH_), jnp.float32),
    )(idx3)
    return out


# speedup vs baseline: 1.4268x; 1.0200x over previous
import jax
import jax.numpy as jnp
from jax.experimental import pallas as pl
from jax.experimental.pallas import tpu as pltpu

DEPTH_ = 1000
ROWS_PER_BLOCK = 128


def _onehot_body(idx_ref, out_ref):
    r, f, _ = idx_ref.shape
    iota = jax.lax.broadcasted_iota(jnp.int32, (r, f, DEPTH_), 2)
    out_ref[...] = (iota == idx_ref[...]).astype(jnp.float32)


def kernel(indices):
    b, f = indices.shape
    idx3 = indices.astype(jnp.int32)[..., None]
    out = pl.pallas_call(
        _onehot_body,
        grid=(b // ROWS_PER_BLOCK,),
        in_specs=[pl.BlockSpec((ROWS_PER_BLOCK, f, 1), lambda i: (i, 0, 0))],
        out_specs=pl.BlockSpec((ROWS_PER_BLOCK, f, DEPTH_), lambda i: (i, 0, 0)),
        out_shape=jax.ShapeDtypeStruct((b, f, DEPTH_), jnp.float32),
    )(idx3)
    return out
